# baseline (speedup 1.0000x reference)
import functools

import jax
import jax.numpy as jnp
from jax import lax
from jax.experimental import pallas as pl
from jax.experimental.pallas import tpu as pltpu
from jax.experimental.pallas import tpu_sc as plsc

NC = 2
NS = 16
L = 16
NW = NC * NS
BATCH = 4096
DIM = 128
BPW = BATCH // NW
CH = 64
NCH = BPW // CH

_MESH = plsc.VectorSubcoreMesh(core_axis_name="c", subcore_axis_name="s")


@functools.partial(
    pl.kernel,
    out_type=jax.ShapeDtypeStruct((NW // 4, 4 * L), jnp.float32),
    mesh=_MESH,
    scratch_types=[
        pltpu.VMEM((BPW,), jnp.int32),
        pltpu.VMEM((BPW, DIM), jnp.float32),
        pltpu.VMEM((BPW, DIM), jnp.float32),
        pltpu.VMEM((L,), jnp.float32),
    ]
    + [pltpu.SemaphoreType.DMA] * (2 * NCH),
)
def _center_loss_partials(inp_hbm, tgt_hbm, cen_hbm, out_hbm,
                          idx_v, rows_v, in_v, acc_v, *sems):
    in_sems, g_sems = sems[:NCH], sems[NCH:]
    wid = lax.axis_index("s") * NC + lax.axis_index("c")
    base = wid * BPW

    in_copies = [
        pltpu.async_copy(
            inp_hbm.at[pl.ds(base + k * CH, CH), :],
            in_v.at[pl.ds(k * CH, CH)], in_sems[k])
        for k in range(NCH)
    ]
    pltpu.sync_copy(tgt_hbm.at[pl.ds(base, BPW)], idx_v)
    copies = []
    for k in range(NCH):
        r = pl.ds(k * CH, CH)
        g = pltpu.async_copy(
            cen_hbm.at[idx_v.at[r]], rows_v.at[r], g_sems[k])
        copies.append((g, in_copies[k]))

    def chunk_rows(k, accs):
        def row_body(i, accs):
            new = []
            for j in range(DIM // L):
                a = in_v[k * CH + i, pl.ds(j * L, L)]
                b = rows_v[k * CH + i, pl.ds(j * L, L)]
                d = a - b
                new.append(accs[j] + d * d)
            return tuple(new)
        return lax.fori_loop(0, CH, row_body, accs)

    accs = tuple(jnp.zeros((L,), jnp.float32) for _ in range(DIM // L))
    for k in range(NCH):
        copies[k][0].wait()
        copies[k][1].wait()
        accs = chunk_rows(k, accs)

    a01, a23 = accs[0] + accs[1], accs[2] + accs[3]
    a45, a67 = accs[4] + accs[5], accs[6] + accs[7]
    acc_v[...] = (a01 + a23) + (a45 + a67)
    pltpu.sync_copy(acc_v, out_hbm.at[wid // 4, pl.ds((wid % 4) * L, L)])


@jax.jit
def kernel(input, target, centers):
    partials = _center_loss_partials(input, target.astype(jnp.int32), centers)
    return jnp.sum(partials) / (2.0 * BATCH)


# speedup vs baseline: 1.0103x; 1.0103x over previous
import functools

import jax
import jax.numpy as jnp
from jax import lax
from jax.experimental import pallas as pl
from jax.experimental.pallas import tpu as pltpu
from jax.experimental.pallas import tpu_sc as plsc

NC = 2
NS = 16
L = 16
NW = NC * NS
BATCH = 4096
DIM = 128
BPW = BATCH // NW
CH = 64
NCH = BPW // CH

_MESH = plsc.VectorSubcoreMesh(
    core_axis_name="c", subcore_axis_name="s", num_cores=NC)


@functools.partial(
    pl.kernel,
    out_type=jax.ShapeDtypeStruct((NW // 4, 4 * L), jnp.float32),
    mesh=_MESH,
    scratch_types=[
        pltpu.VMEM((BPW,), jnp.int32),
        pltpu.VMEM((BPW, DIM), jnp.float32),
        pltpu.VMEM((BPW, DIM), jnp.float32),
        pltpu.VMEM((L,), jnp.float32),
    ]
    + [pltpu.SemaphoreType.DMA] * (2 * NCH),
)
def _center_loss_partials(inp_hbm, tgt_hbm, cen_hbm, out_hbm,
                          idx_v, rows_v, in_v, acc_v, *sems):
    in_sems, g_sems = sems[:NCH], sems[NCH:]
    wid = lax.axis_index("s") * NC + lax.axis_index("c")
    base = wid * BPW

    in_copies = [
        pltpu.async_copy(
            inp_hbm.at[pl.ds(base + k * CH, CH), :],
            in_v.at[pl.ds(k * CH, CH)], in_sems[k])
        for k in range(NCH)
    ]
    pltpu.sync_copy(tgt_hbm.at[pl.ds(base, BPW)], idx_v)
    copies = []
    for k in range(NCH):
        r = pl.ds(k * CH, CH)
        g = pltpu.async_copy(
            cen_hbm.at[idx_v.at[r]], rows_v.at[r], g_sems[k])
        copies.append((g, in_copies[k]))

    def chunk_rows(k, accs):
        def row_body(i, accs):
            new = []
            for j in range(DIM // L):
                a = in_v[k * CH + i, pl.ds(j * L, L)]
                b = rows_v[k * CH + i, pl.ds(j * L, L)]
                d = a - b
                new.append(accs[j] + d * d)
            return tuple(new)
        return lax.fori_loop(0, CH, row_body, accs)

    accs = tuple(jnp.zeros((L,), jnp.float32) for _ in range(DIM // L))
    for k in range(NCH):
        copies[k][0].wait()
        copies[k][1].wait()
        accs = chunk_rows(k, accs)

    a01, a23 = accs[0] + accs[1], accs[2] + accs[3]
    a45, a67 = accs[4] + accs[5], accs[6] + accs[7]
    acc_v[...] = (a01 + a23) + (a45 + a67)
    pltpu.sync_copy(acc_v, out_hbm.at[wid // 4, pl.ds((wid % 4) * L, L)])


@jax.jit
def kernel(input, target, centers):
    partials = _center_loss_partials(input, target.astype(jnp.int32), centers)
    return jnp.sum(partials) / (2.0 * BATCH)
